# Initial kernel scaffold; baseline (speedup 1.0000x reference)
import functools

import jax
import jax.numpy as jnp
from jax import lax
from jax.experimental import pallas as pl
from jax.experimental.pallas import tpu as pltpu
from jax.experimental.pallas import tpu_sc as plsc

VOCAB = 1000000
EMB = 32
B = 16384
L = 50
N = B * L

NC = 2
NS = 16
NW = NC * NS
CH = 128
CPW = N // (NW * CH)

_mesh = plsc.VectorSubcoreMesh(core_axis_name="c", subcore_axis_name="s")


@functools.partial(
    pl.kernel,
    mesh=_mesh,
    out_type=jax.ShapeDtypeStruct((NW, CPW, CH, EMB), jnp.float32),
    scratch_types=[
        pltpu.VMEM((CPW, CH), jnp.int32),
        pltpu.VMEM((2, CH, EMB), jnp.float32),
        pltpu.SemaphoreType.DMA,
    ],
)
def _emb_lookup(idx_hbm, tab_hbm, out_hbm, idx_v, rows_v, gsem):
    wid = lax.axis_index("s") * NC + lax.axis_index("c")
    pltpu.sync_copy(idx_hbm.at[wid], idx_v)

    def body(j, carry):
        pltpu.async_copy(tab_hbm.at[idx_v.at[j]], rows_v.at[0], gsem).wait()
        pltpu.sync_copy(rows_v.at[0], out_hbm.at[wid, j])
        return carry

    lax.fori_loop(0, CPW, body, 0)


def kernel(x, C):
    xw = x.reshape(NW, CPW, CH)
    out = _emb_lookup(xw, C)
    return out.reshape(B, L, EMB)


# speedup vs baseline: 1.1862x; 1.1862x over previous
import functools

import jax
import jax.numpy as jnp
from jax import lax
from jax.experimental import pallas as pl
from jax.experimental.pallas import tpu as pltpu
from jax.experimental.pallas import tpu_sc as plsc

VOCAB = 1000000
EMB = 32
B = 16384
L = 50
N = B * L

NC = 2
NS = 16
NW = NC * NS
CH = 128
CPW = N // (NW * CH)

_mesh = plsc.VectorSubcoreMesh(core_axis_name="c", subcore_axis_name="s")


@functools.partial(
    pl.kernel,
    mesh=_mesh,
    compiler_params=pltpu.CompilerParams(use_tc_tiling_on_sc=False),
    out_type=jax.ShapeDtypeStruct((NW, CPW, CH, EMB), jnp.float32),
    scratch_types=[
        pltpu.VMEM((CPW, CH), jnp.int32),
        pltpu.VMEM((2, CH, EMB), jnp.float32),
        pltpu.SemaphoreType.DMA,
    ],
)
def _emb_lookup(idx_hbm, tab_hbm, out_hbm, idx_v, rows_v, gsem):
    wid = lax.axis_index("s") * NC + lax.axis_index("c")
    pltpu.sync_copy(idx_hbm.at[wid], idx_v)

    def body(j, carry):
        pltpu.async_copy(tab_hbm.at[idx_v.at[j]], rows_v.at[0], gsem).wait()
        pltpu.sync_copy(rows_v.at[0], out_hbm.at[wid, j])
        return carry

    lax.fori_loop(0, CPW, body, 0)


def kernel(x, C):
    xw = x.reshape(NW, CPW, CH)
    out = _emb_lookup(xw, C)
    return out.reshape(B, L, EMB)


# speedup vs baseline: 1.3087x; 1.1033x over previous
import functools

import jax
import jax.numpy as jnp
from jax import lax
from jax.experimental import pallas as pl
from jax.experimental.pallas import tpu as pltpu
from jax.experimental.pallas import tpu_sc as plsc

VOCAB = 1000000
EMB = 32
B = 16384
L = 50
N = B * L

NC = 2
NS = 16
NW = NC * NS
CH = 128
CPW = N // (NW * CH)
K = 5
G = CPW // K

_mesh = plsc.VectorSubcoreMesh(core_axis_name="c", subcore_axis_name="s")


@functools.partial(
    pl.kernel,
    mesh=_mesh,
    compiler_params=pltpu.CompilerParams(use_tc_tiling_on_sc=False),
    out_type=jax.ShapeDtypeStruct((NW, CPW, CH, EMB), jnp.float32),
    scratch_types=[
        pltpu.VMEM((CPW, CH), jnp.int32),
        pltpu.VMEM((2, K, CH, EMB), jnp.float32),
        pltpu.SemaphoreType.DMA((2,)),
        pltpu.SemaphoreType.DMA((2,)),
    ],
)
def _emb_lookup(idx_hbm, tab_hbm, out_hbm, idx_v, rows_v, gsem, osem):
    wid = lax.axis_index("s") * NC + lax.axis_index("c")
    pltpu.sync_copy(idx_hbm.at[wid], idx_v)

    def fire_gathers(g, p):
        for b in range(K):
            pltpu.async_copy(
                tab_hbm.at[idx_v.at[g * K + b]], rows_v.at[p, b], gsem.at[p])

    def drain_gathers(p):
        for _ in range(K):
            pltpu.make_async_copy(
                tab_hbm.at[idx_v.at[0]], rows_v.at[p, 0], gsem.at[p]).wait()

    def fire_outs(g, p):
        for b in range(K):
            pltpu.async_copy(
                rows_v.at[p, b], out_hbm.at[wid, g * K + b], osem.at[p])

    def drain_outs(p):
        for _ in range(K):
            pltpu.make_async_copy(
                rows_v.at[p, 0], out_hbm.at[wid, 0], osem.at[p]).wait()

    fire_gathers(0, 0)
    fire_gathers(1, 1)
    drain_gathers(0)
    fire_outs(0, 0)

    def body(g, carry):
        p = lax.rem(g, 2)
        q = 1 - p
        drain_outs(p)
        fire_gathers(g, p)
        drain_gathers(q)
        fire_outs(g - 1, q)
        return carry

    lax.fori_loop(2, G, body, 0)

    drain_gathers((G - 1) % 2)
    fire_outs(G - 1, (G - 1) % 2)
    drain_outs((G - 2) % 2)
    drain_outs((G - 1) % 2)


def kernel(x, C):
    xw = x.reshape(NW, CPW, CH)
    out = _emb_lookup(xw, C)
    return out.reshape(B, L, EMB)


# speedup vs baseline: 1.7992x; 1.3748x over previous
import functools

import jax
import jax.numpy as jnp
from jax import lax
from jax.experimental import pallas as pl
from jax.experimental.pallas import tpu as pltpu
from jax.experimental.pallas import tpu_sc as plsc

VOCAB = 1000000
EMB = 32
B = 16384
L = 50

NC = 2
NS = 16
NW = NC * NS
RPW = B // NW
K = 8
G = RPW // K

_mesh = plsc.VectorSubcoreMesh(core_axis_name="c", subcore_axis_name="s")


@functools.partial(
    pl.kernel,
    mesh=_mesh,
    compiler_params=pltpu.CompilerParams(use_tc_tiling_on_sc=False),
    out_type=jax.ShapeDtypeStruct((B, L, EMB), jnp.float32),
    scratch_types=[
        pltpu.VMEM((RPW, L), jnp.int32),
        pltpu.VMEM((2, K, L, EMB), jnp.float32),
        pltpu.SemaphoreType.DMA((2,)),
        pltpu.SemaphoreType.DMA((2,)),
    ],
)
def _emb_lookup(idx_hbm, tab_hbm, out_hbm, idx_v, rows_v, gsem, osem):
    wid = lax.axis_index("s") * NC + lax.axis_index("c")
    base = wid * RPW
    pltpu.sync_copy(idx_hbm.at[pl.ds(base, RPW)], idx_v)

    def fire_gathers(g, p):
        for b in range(K):
            pltpu.async_copy(
                tab_hbm.at[idx_v.at[g * K + b]], rows_v.at[p, b], gsem.at[p])

    def drain_gathers(p):
        for _ in range(K):
            pltpu.make_async_copy(
                tab_hbm.at[idx_v.at[0]], rows_v.at[p, 0], gsem.at[p]).wait()

    def fire_outs(g, p):
        for b in range(K):
            pltpu.async_copy(
                rows_v.at[p, b], out_hbm.at[base + g * K + b], osem.at[p])

    def drain_outs(p):
        for _ in range(K):
            pltpu.make_async_copy(
                rows_v.at[p, 0], out_hbm.at[0], osem.at[p]).wait()

    fire_gathers(0, 0)
    fire_gathers(1, 1)
    drain_gathers(0)
    fire_outs(0, 0)

    def body(g, carry):
        p = lax.rem(g, 2)
        q = 1 - p
        drain_outs(p)
        fire_gathers(g, p)
        drain_gathers(q)
        fire_outs(g - 1, q)
        return carry

    lax.fori_loop(2, G, body, 0)

    drain_gathers((G - 1) % 2)
    fire_outs(G - 1, (G - 1) % 2)
    drain_outs((G - 2) % 2)
    drain_outs((G - 1) % 2)


def kernel(x, C):
    return _emb_lookup(x, C)
